# Initial kernel scaffold; baseline (speedup 1.0000x reference)
import jax
import jax.numpy as jnp
from jax.experimental import pallas as pl


def kernel(A, x):
    raise NotImplementedError("write your pallas kernel here")


# speedup vs baseline: 1.2767x; 1.2767x over previous
import functools

import jax
import jax.numpy as jnp
from jax import lax
from jax.experimental import pallas as pl
from jax.experimental.pallas import tpu as pltpu
from jax.experimental.pallas import tpu_sc as plsc

_B, _N, _P, _GS = 16, 2048, 128, 512
_CH = 256
_NTILES = 32
_RPT = (_B * _GS) // _NTILES
_K = 8


def _topk_body(vrow_ref, vcol_ref, vo_ref, gvo_ref):
    b = pl.program_id(0)
    v_row = vrow_ref[0]
    v_col = vcol_ref[0]
    lane = lax.broadcasted_iota(jnp.int32, (1, _N), 1)
    chunks = []
    for c in range(_N // _CH):
        vj = v_col[c * _CH:(c + 1) * _CH, :]
        js = lax.broadcasted_iota(jnp.int32, (_CH, 1), 0) + c * _CH
        first = (vj > v_row) | ((vj == v_row) & (js < lane))
        chunks.append(
            (_N - 1.0) - jnp.sum(first.astype(jnp.float32), axis=1, keepdims=True))
    ranks_col = jnp.concatenate(chunks, axis=0)
    r_row = lax.broadcasted_iota(jnp.int32, (1, _GS), 1).astype(jnp.float32)
    onehot_t = (ranks_col == r_row).astype(jnp.float32)
    idx_col = lax.broadcasted_iota(jnp.int32, (_N, 1), 0).astype(jnp.float32)
    vo = jnp.sum(onehot_t * idx_col, axis=0, keepdims=True).astype(jnp.int32)
    vo_ref[0] = vo
    gvo_ref[0] = vo + b * _N


def _topk_tc(xl_row, xl_col):
    return pl.pallas_call(
        _topk_body,
        grid=(_B,),
        in_specs=[
            pl.BlockSpec((1, 1, _N), lambda b: (b, 0, 0)),
            pl.BlockSpec((1, _N, 1), lambda b: (b, 0, 0)),
        ],
        out_specs=(
            pl.BlockSpec((1, 1, _GS), lambda b: (b, 0, 0)),
            pl.BlockSpec((1, 1, _GS), lambda b: (b, 0, 0)),
        ),
        out_shape=(
            jax.ShapeDtypeStruct((_B, 1, _GS), jnp.int32),
            jax.ShapeDtypeStruct((_B, 1, _GS), jnp.int32),
        ),
    )(xl_row, xl_col)


def _sc_gather(a2, x2, vo, gvo):
    mesh = plsc.VectorSubcoreMesh(core_axis_name="c", subcore_axis_name="s")

    @functools.partial(
        pl.kernel,
        mesh=mesh,
        out_type=(
            jax.ShapeDtypeStruct((_B * _GS, _GS), jnp.float32),
            jax.ShapeDtypeStruct((_B * _GS, _P), jnp.float32),
        ),
        scratch_types=[
            pltpu.VMEM((_GS,), jnp.int32),
            pltpu.VMEM((_RPT,), jnp.int32),
            pltpu.VMEM((_RPT, _P), jnp.float32),
            pltpu.VMEM((_K, _N), jnp.float32),
            pltpu.VMEM((_N,), jnp.float32),
            pltpu.VMEM((_GS,), jnp.float32),
            pltpu.SemaphoreType.DMA,
        ],
        compiler_params=pltpu.CompilerParams(needs_layout_passes=False),
    )
    def sc_k(a_hbm, x_hbm, vo_hbm, gvo_hbm, at2_hbm, xg_hbm,
             ior_v, gior_v, xg_v, rows_v, row1d, out1d, sem):
        wid = lax.axis_index("s") * 2 + lax.axis_index("c")
        b = wid // 2
        h = wid % 2
        r0 = b * _GS + h * _RPT
        pltpu.sync_copy(vo_hbm.at[pl.ds(b * _GS, _GS)], ior_v)
        pltpu.sync_copy(gvo_hbm.at[pl.ds(r0, _RPT)], gior_v)

        for i in range(_RPT // 128):
            pltpu.async_copy(
                x_hbm.at[gior_v.at[pl.ds(i * 128, 128)]],
                xg_v.at[pl.ds(i * 128, 128)], sem).wait()
        pltpu.sync_copy(xg_v, xg_hbm.at[pl.ds(r0, _RPT)])

        def step(t, carry):
            pltpu.async_copy(
                a_hbm.at[gior_v.at[pl.ds(t * _K, _K)]], rows_v, sem).wait()
            for j in range(_K):
                jv = jnp.full((16,), j, dtype=jnp.int32)
                for c in range(_GS // 16):
                    colv = ior_v[pl.ds(c * 16, 16)]
                    out1d[pl.ds(c * 16, 16)] = plsc.load_gather(
                        rows_v, [jv, colv])
                pltpu.sync_copy(out1d, at2_hbm.at[r0 + t * _K + j])
            return carry

        lax.fori_loop(0, _RPT // _K, step, 0)

    return sc_k(a2, x2, vo, gvo)


def kernel(A, x):
    xl = x[:, :, -1]
    vo3, gvo3 = _topk_tc(xl[:, None, :], xl[:, :, None])
    vo = vo3.reshape(_B * _GS)
    gvo = gvo3.reshape(_B * _GS)
    at2, xg = _sc_gather(
        A.reshape(_B * _N, _N), x.reshape(_B * _N, _P), vo, gvo)
    return at2.reshape(_B, _GS, _GS), xg.reshape(_B, _GS, _P)


# speedup vs baseline: 1.7128x; 1.3416x over previous
import functools

import jax
import jax.numpy as jnp
from jax import lax
from jax.experimental import pallas as pl
from jax.experimental.pallas import tpu as pltpu
from jax.experimental.pallas import tpu_sc as plsc

_B, _N, _P, _GS = 16, 2048, 128, 512
_CH = 256
_NTILES = 32
_RPT = (_B * _GS) // _NTILES
_K = 8


def _topk_body(vrow_ref, vcol_ref, vo_ref, gvo_ref):
    b = pl.program_id(0)
    v_row = vrow_ref[0]
    v_col = vcol_ref[0]
    lane = lax.broadcasted_iota(jnp.int32, (1, _N), 1)
    chunks = []
    for c in range(_N // _CH):
        vj = v_col[c * _CH:(c + 1) * _CH, :]
        js = lax.broadcasted_iota(jnp.int32, (_CH, 1), 0) + c * _CH
        first = (vj > v_row) | ((vj == v_row) & (js < lane))
        chunks.append(
            (_N - 1.0) - jnp.sum(first.astype(jnp.float32), axis=1, keepdims=True))
    ranks_col = jnp.concatenate(chunks, axis=0)
    r_row = lax.broadcasted_iota(jnp.int32, (1, _GS), 1).astype(jnp.float32)
    onehot_t = (ranks_col == r_row).astype(jnp.float32)
    idx_col = lax.broadcasted_iota(jnp.int32, (_N, 1), 0).astype(jnp.float32)
    vo = jnp.sum(onehot_t * idx_col, axis=0, keepdims=True).astype(jnp.int32)
    vo_ref[0] = vo
    gvo_ref[0] = vo + b * _N


def _topk_tc(xl_row, xl_col):
    return pl.pallas_call(
        _topk_body,
        grid=(_B,),
        in_specs=[
            pl.BlockSpec((1, 1, _N), lambda b: (b, 0, 0)),
            pl.BlockSpec((1, _N, 1), lambda b: (b, 0, 0)),
        ],
        out_specs=(
            pl.BlockSpec((1, 1, _GS), lambda b: (b, 0, 0)),
            pl.BlockSpec((1, 1, _GS), lambda b: (b, 0, 0)),
        ),
        out_shape=(
            jax.ShapeDtypeStruct((_B, 1, _GS), jnp.int32),
            jax.ShapeDtypeStruct((_B, 1, _GS), jnp.int32),
        ),
    )(xl_row, xl_col)


def _sc_gather(a2, x2, vo, gvo):
    mesh = plsc.VectorSubcoreMesh(core_axis_name="c", subcore_axis_name="s")

    @functools.partial(
        pl.kernel,
        mesh=mesh,
        out_type=(
            jax.ShapeDtypeStruct((_B * _GS, _GS), jnp.float32),
            jax.ShapeDtypeStruct((_B * _GS, _P), jnp.float32),
        ),
        scratch_types=[
            pltpu.VMEM((_GS,), jnp.int32),
            pltpu.VMEM((_RPT,), jnp.int32),
            pltpu.VMEM((_RPT, _P), jnp.float32),
            pltpu.VMEM((_K, _N), jnp.float32),
            pltpu.VMEM((_K, _N), jnp.float32),
            pltpu.VMEM((_K, _GS), jnp.float32),
            pltpu.VMEM((_K, _GS), jnp.float32),
            pltpu.SemaphoreType.DMA,
            pltpu.SemaphoreType.DMA,
            pltpu.SemaphoreType.DMA,
            pltpu.SemaphoreType.DMA,
            pltpu.SemaphoreType.DMA,
        ],
        compiler_params=pltpu.CompilerParams(needs_layout_passes=False),
    )
    def sc_k(a_hbm, x_hbm, vo_hbm, gvo_hbm, at2_hbm, xg_hbm,
             ior_v, gior_v, xg_v, rows_a, rows_b, out_a, out_b,
             sem_a, sem_b, sem_oa, sem_ob, sem_xg):
        wid = lax.axis_index("s") * 2 + lax.axis_index("c")
        b = wid // 2
        h = wid % 2
        r0 = b * _GS + h * _RPT
        pltpu.sync_copy(vo_hbm.at[pl.ds(b * _GS, _GS)], ior_v)
        pltpu.sync_copy(gvo_hbm.at[pl.ds(r0, _RPT)], gior_v)

        for i in range(_RPT // 128):
            pltpu.async_copy(
                x_hbm.at[gior_v.at[pl.ds(i * 128, 128)]],
                xg_v.at[pl.ds(i * 128, 128)], sem_xg)

        def select_cols(rows, out):
            for c in range(_GS // 16):
                colv = ior_v[pl.ds(c * 16, 16)]
                for j in range(_K):
                    jv = jnp.full((16,), j, dtype=jnp.int32)
                    out[j, pl.ds(c * 16, 16)] = plsc.load_gather(
                        rows, [jv, colv])

        def fetch(t, rows, sem):
            pltpu.async_copy(
                a_hbm.at[gior_v.at[pl.ds(t * _K, _K)]], rows, sem)

        def drain_in(rows, sem):
            pltpu.make_async_copy(a_hbm.at[pl.ds(0, _K)], rows, sem).wait()

        def drain_out(out, sem):
            pltpu.make_async_copy(at2_hbm.at[pl.ds(0, _K)], out, sem).wait()

        nchunks = _RPT // _K
        fetch(0, rows_a, sem_a)

        def body(tt, carry):
            t0 = 2 * tt
            fetch(t0 + 1, rows_b, sem_b)
            drain_in(rows_a, sem_a)

            @pl.when(tt > 0)
            def _():
                drain_out(out_a, sem_oa)

            select_cols(rows_a, out_a)
            pltpu.async_copy(out_a, at2_hbm.at[pl.ds(r0 + t0 * _K, _K)],
                             sem_oa)

            @pl.when(tt < nchunks // 2 - 1)
            def _():
                fetch(t0 + 2, rows_a, sem_a)

            drain_in(rows_b, sem_b)

            @pl.when(tt > 0)
            def _():
                drain_out(out_b, sem_ob)

            select_cols(rows_b, out_b)
            pltpu.async_copy(out_b, at2_hbm.at[pl.ds(r0 + (t0 + 1) * _K, _K)],
                             sem_ob)
            return carry

        lax.fori_loop(0, nchunks // 2, body, 0)
        drain_out(out_a, sem_oa)
        drain_out(out_b, sem_ob)

        for i in range(_RPT // 128):
            pltpu.make_async_copy(
                x_hbm.at[pl.ds(0, 128)],
                xg_v.at[pl.ds(i * 128, 128)], sem_xg).wait()
        pltpu.sync_copy(xg_v, xg_hbm.at[pl.ds(r0, _RPT)])

    return sc_k(a2, x2, vo, gvo)


def kernel(A, x):
    xl = x[:, :, -1]
    vo3, gvo3 = _topk_tc(xl[:, None, :], xl[:, :, None])
    vo = vo3.reshape(_B * _GS)
    gvo = gvo3.reshape(_B * _GS)
    at2, xg = _sc_gather(
        A.reshape(_B * _N, _N), x.reshape(_B * _N, _P), vo, gvo)
    return at2.reshape(_B, _GS, _GS), xg.reshape(_B, _GS, _P)


# speedup vs baseline: 2.0422x; 1.1923x over previous
import functools

import jax
import jax.numpy as jnp
from jax import lax
from jax.experimental import pallas as pl
from jax.experimental.pallas import tpu as pltpu
from jax.experimental.pallas import tpu_sc as plsc

_B, _N, _P, _GS = 16, 2048, 128, 512
_CH = 256
_NTILES = 32
_RPT = (_B * _GS) // _NTILES
_K = 8


def _topk_body(x_ref, vo_ref, gvo_ref):
    b = pl.program_id(0)
    v_col = x_ref[0, :, _P - 1:_P]
    v_row = jnp.reshape(v_col, (_N // 128, 128))
    chunks = []
    for c in range(_N // _CH):
        vj = v_col[c * _CH:(c + 1) * _CH, :]
        js = lax.broadcasted_iota(jnp.int32, (_CH, 1), 0) + c * _CH
        acc = jnp.zeros((_CH, 128), jnp.float32)
        for ci in range(_N // 128):
            vi = v_row[ci:ci + 1, :]
            lane = lax.broadcasted_iota(jnp.int32, (1, 128), 1) + ci * 128
            first = (vj > vi) | ((vj == vi) & (js < lane))
            acc = acc + first.astype(jnp.float32)
        cnt = jnp.sum(acc, axis=1, keepdims=True)
        chunks.append((_N - 1.0) - cnt)
    ranks_col = jnp.concatenate(chunks, axis=0)
    r_row = lax.broadcasted_iota(jnp.int32, (1, _GS), 1).astype(jnp.float32)
    onehot_t = (ranks_col == r_row).astype(jnp.float32)
    idx_col = lax.broadcasted_iota(jnp.int32, (_N, 1), 0).astype(jnp.float32)
    vo = jnp.sum(onehot_t * idx_col, axis=0, keepdims=True).astype(jnp.int32)
    vo_ref[0] = vo
    gvo_ref[0] = vo + b * _N


def _topk_tc(x):
    return pl.pallas_call(
        _topk_body,
        grid=(_B,),
        in_specs=[
            pl.BlockSpec((1, _N, _P), lambda b: (b, 0, 0)),
        ],
        out_specs=(
            pl.BlockSpec((1, 1, _GS), lambda b: (b, 0, 0)),
            pl.BlockSpec((1, 1, _GS), lambda b: (b, 0, 0)),
        ),
        out_shape=(
            jax.ShapeDtypeStruct((_B, 1, _GS), jnp.int32),
            jax.ShapeDtypeStruct((_B, 1, _GS), jnp.int32),
        ),
    )(x)


def _sc_gather(a2, x2, vo, gvo):
    mesh = plsc.VectorSubcoreMesh(core_axis_name="c", subcore_axis_name="s")

    @functools.partial(
        pl.kernel,
        mesh=mesh,
        out_type=(
            jax.ShapeDtypeStruct((_B * _GS, _GS), jnp.float32),
            jax.ShapeDtypeStruct((_B * _GS, _P), jnp.float32),
        ),
        scratch_types=[
            pltpu.VMEM((_GS,), jnp.int32),
            pltpu.VMEM((_RPT,), jnp.int32),
            pltpu.VMEM((_RPT, _P), jnp.float32),
            pltpu.VMEM((_K, _N), jnp.float32),
            pltpu.VMEM((_K, _N), jnp.float32),
            pltpu.VMEM((_K, _GS), jnp.float32),
            pltpu.VMEM((_K, _GS), jnp.float32),
            pltpu.SemaphoreType.DMA,
            pltpu.SemaphoreType.DMA,
            pltpu.SemaphoreType.DMA,
            pltpu.SemaphoreType.DMA,
            pltpu.SemaphoreType.DMA,
        ],
        compiler_params=pltpu.CompilerParams(needs_layout_passes=False),
    )
    def sc_k(a_hbm, x_hbm, vo_hbm, gvo_hbm, at2_hbm, xg_hbm,
             ior_v, gior_v, xg_v, rows_a, rows_b, out_a, out_b,
             sem_a, sem_b, sem_oa, sem_ob, sem_xg):
        wid = lax.axis_index("s") * 2 + lax.axis_index("c")
        b = wid // 2
        h = wid % 2
        r0 = b * _GS + h * _RPT
        pltpu.sync_copy(vo_hbm.at[pl.ds(b * _GS, _GS)], ior_v)
        pltpu.sync_copy(gvo_hbm.at[pl.ds(r0, _RPT)], gior_v)

        for i in range(_RPT // 128):
            pltpu.async_copy(
                x_hbm.at[gior_v.at[pl.ds(i * 128, 128)]],
                xg_v.at[pl.ds(i * 128, 128)], sem_xg)

        def select_cols(rows, out):
            for c in range(_GS // 16):
                colv = ior_v[pl.ds(c * 16, 16)]
                for j in range(_K):
                    jv = jnp.full((16,), j, dtype=jnp.int32)
                    out[j, pl.ds(c * 16, 16)] = plsc.load_gather(
                        rows, [jv, colv])

        def fetch(t, rows, sem):
            pltpu.async_copy(
                a_hbm.at[gior_v.at[pl.ds(t * _K, _K)]], rows, sem)

        def drain_in(rows, sem):
            pltpu.make_async_copy(a_hbm.at[pl.ds(0, _K)], rows, sem).wait()

        def drain_out(out, sem):
            pltpu.make_async_copy(at2_hbm.at[pl.ds(0, _K)], out, sem).wait()

        nchunks = _RPT // _K
        fetch(0, rows_a, sem_a)

        def body(tt, carry):
            t0 = 2 * tt
            fetch(t0 + 1, rows_b, sem_b)
            drain_in(rows_a, sem_a)

            @pl.when(tt > 0)
            def _():
                drain_out(out_a, sem_oa)

            select_cols(rows_a, out_a)
            pltpu.async_copy(out_a, at2_hbm.at[pl.ds(r0 + t0 * _K, _K)],
                             sem_oa)

            @pl.when(tt < nchunks // 2 - 1)
            def _():
                fetch(t0 + 2, rows_a, sem_a)

            drain_in(rows_b, sem_b)

            @pl.when(tt > 0)
            def _():
                drain_out(out_b, sem_ob)

            select_cols(rows_b, out_b)
            pltpu.async_copy(out_b, at2_hbm.at[pl.ds(r0 + (t0 + 1) * _K, _K)],
                             sem_ob)
            return carry

        lax.fori_loop(0, nchunks // 2, body, 0)
        drain_out(out_a, sem_oa)
        drain_out(out_b, sem_ob)

        for i in range(_RPT // 128):
            pltpu.make_async_copy(
                x_hbm.at[pl.ds(0, 128)],
                xg_v.at[pl.ds(i * 128, 128)], sem_xg).wait()
        pltpu.sync_copy(xg_v, xg_hbm.at[pl.ds(r0, _RPT)])

    return sc_k(a2, x2, vo, gvo)


def kernel(A, x):
    vo3, gvo3 = _topk_tc(x)
    vo = vo3.reshape(_B * _GS)
    gvo = gvo3.reshape(_B * _GS)
    at2, xg = _sc_gather(
        A.reshape(_B * _N, _N), x.reshape(_B * _N, _P), vo, gvo)
    return at2.reshape(_B, _GS, _GS), xg.reshape(_B, _GS, _P)
